# Initial kernel scaffold; baseline (speedup 1.0000x reference)
import jax
import jax.numpy as jnp
from jax.experimental import pallas as pl


def kernel(x, edge_index, W1, b1, W2, b2, Wxz, Whz, bz, Wxr, Whr, br, Wxh, Whh, bh, W3, b3, W4, b4):
    raise NotImplementedError("write your pallas kernel here")


# speedup vs baseline: 9.6548x; 9.6548x over previous
import functools

import jax
import jax.numpy as jnp
from jax import lax
from jax.experimental import pallas as pl
from jax.experimental.pallas import tpu as pltpu
from jax.experimental.pallas import tpu_sc as plsc

_N = 10000
_E = 320000
_D = 128
_H = 256
_CHUNK = 128
_NSUB = 16
_EP = 327680
_NCHUNKS = _EP // _CHUNK
_ACC_ROWS = 10240
_RPT = _ACC_ROWS // _NSUB
_G = 16


def _make_sc_agg(split_edges: bool, n_tables: int):
    chunks_per_core = _NCHUNKS // 2 if split_edges else _NCHUNKS
    cpt = chunks_per_core // _NSUB
    n_groups = cpt // _G
    mesh = plsc.VectorSubcoreMesh(core_axis_name="c", subcore_axis_name="s")

    @functools.partial(
        pl.kernel,
        mesh=mesh,
        out_type=jax.ShapeDtypeStruct((2, _N, _D), jnp.float32),
        scratch_types=[
            pltpu.VMEM((2, _G, _CHUNK), jnp.int32),
            pltpu.VMEM((2, _G, _CHUNK), jnp.int32),
            pltpu.VMEM((2, _CHUNK, _D), jnp.float32),
            pltpu.VMEM_SHARED((_ACC_ROWS, _D), jnp.float32),
            pltpu.SemaphoreType.DMA((2,)),
            pltpu.SemaphoreType.DMA,
        ],
    )
    def agg(table, src2d, dst2d, out, srcb, dstb, rows, acc, gsem, isem):
        c = lax.axis_index("c")
        s = lax.axis_index("s")
        tix = 0 if split_edges else c
        chunk0 = s * cpt + (c * chunks_per_core if split_edges else 0)

        def zero_body(i, carry):
            r = i // 8
            col = (i % 8) * 16
            rows[0, r, pl.ds(col, 16)] = jnp.zeros((16,), jnp.float32)
            return carry

        lax.fori_loop(0, _CHUNK * 8, zero_body, 0)
        for j in range(_RPT // _CHUNK):
            pltpu.sync_copy(rows.at[0],
                            acc.at[pl.ds(s * _RPT + j * _CHUNK, _CHUNK)])

        def start_idx(g):
            gb = lax.rem(g, 2)
            pltpu.async_copy(src2d.at[pl.ds(chunk0 + g * _G, _G)],
                             srcb.at[gb], isem)
            pltpu.async_copy(dst2d.at[pl.ds(chunk0 + g * _G, _G)],
                             dstb.at[gb], isem)

        def wait_idx(g):
            gb = lax.rem(g, 2)
            pltpu.make_async_copy(src2d.at[pl.ds(chunk0 + g * _G, _G)],
                                  srcb.at[gb], isem).wait()
            pltpu.make_async_copy(dst2d.at[pl.ds(chunk0 + g * _G, _G)],
                                  dstb.at[gb], isem).wait()

        def start_gather(i, b):
            g = i // _G
            pltpu.async_copy(
                table.at[tix].at[srcb.at[lax.rem(g, 2), i - g * _G]],
                rows.at[b], gsem.at[b])

        def wait_gather(i, b):
            g = i // _G
            pltpu.make_async_copy(
                table.at[tix].at[srcb.at[lax.rem(g, 2), i - g * _G]],
                rows.at[b], gsem.at[b]).wait()

        start_idx(0)
        wait_idx(0)
        start_gather(0, 0)

        plsc.subcore_barrier()

        def body(i, carry):
            g = i // _G
            j = i - g * _G
            b = lax.rem(i, 2)

            @pl.when((j == 0) & (g + 1 < n_groups))
            def _():
                start_idx(g + 1)

            nxt = i + 1

            @pl.when(nxt < cpt)
            def _():
                @pl.when(j == _G - 1)
                def _():
                    wait_idx(g + 1)

                start_gather(nxt, 1 - b)

            wait_gather(i, b)
            ng = i // _G
            pltpu.sync_copy(rows.at[b],
                            acc.at[dstb.at[lax.rem(ng, 2), i - ng * _G]],
                            add=True)
            return carry

        lax.fori_loop(0, cpt, body, 0)

        plsc.subcore_barrier()

        @pl.when(s < _NSUB - 1)
        def _():
            r0 = s * _RPT
            pltpu.sync_copy(acc.at[pl.ds(r0, _RPT)],
                            out.at[c, pl.ds(r0, _RPT)])

        @pl.when(s == _NSUB - 1)
        def _():
            r0 = (_NSUB - 1) * _RPT
            nlast = _N - r0
            pltpu.sync_copy(acc.at[pl.ds(r0, nlast)],
                            out.at[c, pl.ds(r0, nlast)])

    return agg


_agg_edge_split = _make_sc_agg(split_edges=True, n_tables=1)
_agg_feat_split = _make_sc_agg(split_edges=False, n_tables=2)


_BLK = 1000


def _h1_body(x_ref, p_ref, w_ref, b_ref, o_ref):
    u = x_ref[...] + p_ref[0] + p_ref[1]
    h = jnp.dot(u, w_ref[...], preferred_element_type=jnp.float32) + b_ref[...]
    h = jnp.maximum(h, 0.0)
    o_ref[0] = h[:, :_D]
    o_ref[1] = h[:, _D:]


def _tc_h1(x, p, w1, b1):
    return pl.pallas_call(
        _h1_body,
        grid=(_N // _BLK,),
        in_specs=[
            pl.BlockSpec((_BLK, _D), lambda i: (i, 0)),
            pl.BlockSpec((2, _BLK, _D), lambda i: (0, i, 0)),
            pl.BlockSpec((_D, _H), lambda i: (0, 0)),
            pl.BlockSpec((1, _H), lambda i: (0, 0)),
        ],
        out_specs=pl.BlockSpec((2, _BLK, _D), lambda i: (0, i, 0)),
        out_shape=jax.ShapeDtypeStruct((2, _N, _D), jnp.float32),
    )(x, p, w1, b1)


def _z_body(h_ref, a_ref, w2_ref, b2_ref, z_ref, f_ref):
    ua = h_ref[0] + a_ref[0]
    ub = h_ref[1] + a_ref[1]
    z = (jnp.dot(ua, w2_ref[0], preferred_element_type=jnp.float32)
         + jnp.dot(ub, w2_ref[1], preferred_element_type=jnp.float32)
         + b2_ref[...])
    z_ref[...] = z

    @pl.when(pl.program_id(0) == 0)
    def _():
        f_ref[...] = jnp.zeros_like(f_ref)

    f_ref[...] += jnp.sum(z, axis=0, keepdims=True)


def _tc_z(h1pair, a2, w2, b2):
    return pl.pallas_call(
        _z_body,
        grid=(_N // _BLK,),
        in_specs=[
            pl.BlockSpec((2, _BLK, _D), lambda i: (0, i, 0)),
            pl.BlockSpec((2, _BLK, _D), lambda i: (0, i, 0)),
            pl.BlockSpec((2, _D, _D), lambda i: (0, 0, 0)),
            pl.BlockSpec((1, _D), lambda i: (0, 0)),
        ],
        out_specs=[
            pl.BlockSpec((_BLK, _D), lambda i: (i, 0)),
            pl.BlockSpec((1, _D), lambda i: (0, 0)),
        ],
        out_shape=[
            jax.ShapeDtypeStruct((_N, _D), jnp.float32),
            jax.ShapeDtypeStruct((1, _D), jnp.float32),
        ],
    )(h1pair, a2, w2, b2)


def _tail_body(z_ref, f_ref, wxz_ref, bz_ref, wxh_ref, bh_ref,
               w3_ref, b3_ref, w4_ref, b4_ref, o_ref, zs_ref):
    @pl.when(pl.program_id(0) == 0)
    def _():
        f = f_ref[...] * (1.0 / _N)
        zg = jax.nn.sigmoid(
            jnp.dot(f, wxz_ref[...], preferred_element_type=jnp.float32)
            + bz_ref[...])
        hh = jnp.tanh(
            jnp.dot(f, wxh_ref[...], preferred_element_type=jnp.float32)
            + bh_ref[...])
        ht = (1.0 - zg) * hh
        e = jnp.maximum(
            jnp.dot(ht, w3_ref[...], preferred_element_type=jnp.float32)
            + b3_ref[...], 0.0)
        zs_ref[...] = (jnp.dot(e, w4_ref[...], preferred_element_type=jnp.float32)
                       + b4_ref[...])

    o_ref[...] = lax.dot_general(
        z_ref[...], zs_ref[...],
        dimension_numbers=(((1,), (1,)), ((), ())),
        preferred_element_type=jnp.float32)


def _tc_tail(z, f, wxz, bz, wxh, bh, w3, b3, w4, b4):
    return pl.pallas_call(
        _tail_body,
        grid=(_N // _BLK,),
        in_specs=[
            pl.BlockSpec((_BLK, _D), lambda i: (i, 0)),
            pl.BlockSpec((1, _D), lambda i: (0, 0)),
            pl.BlockSpec((_D, _H), lambda i: (0, 0)),
            pl.BlockSpec((1, _H), lambda i: (0, 0)),
            pl.BlockSpec((_D, _H), lambda i: (0, 0)),
            pl.BlockSpec((1, _H), lambda i: (0, 0)),
            pl.BlockSpec((_H, _H), lambda i: (0, 0)),
            pl.BlockSpec((1, _H), lambda i: (0, 0)),
            pl.BlockSpec((_H, _D), lambda i: (0, 0)),
            pl.BlockSpec((1, _D), lambda i: (0, 0)),
        ],
        out_specs=pl.BlockSpec((_BLK, 1), lambda i: (i, 0)),
        out_shape=jax.ShapeDtypeStruct((_N, 1), jnp.float32),
        scratch_shapes=[pltpu.VMEM((1, _D), jnp.float32)],
    )(z, f, wxz, bz, wxh, bh, w3, b3, w4, b4)


def kernel(x, edge_index, W1, b1, W2, b2, Wxz, Whz, bz, Wxr, Whr, br,
           Wxh, Whh, bh, W3, b3, W4, b4):
    pad = _EP - _E
    ar = jnp.arange(pad, dtype=jnp.int32)
    src_pad = (ar * 37) % _N
    dst_pad = _N + (ar % (_ACC_ROWS - _N))
    src2d = jnp.concatenate([edge_index[0], src_pad]).reshape(_NCHUNKS, _CHUNK)
    dst2d = jnp.concatenate([edge_index[1], dst_pad]).reshape(_NCHUNKS, _CHUNK)

    p1 = _agg_edge_split(x.reshape(1, _N, _D), src2d, dst2d)
    h1pair = _tc_h1(x, p1, W1, b1.reshape(1, _H))
    a2 = _agg_feat_split(h1pair, src2d, dst2d)
    z, f = _tc_z(h1pair, a2, W2.reshape(2, _D, _D), b2.reshape(1, _D))
    logits = _tc_tail(z, f, Wxz, bz.reshape(1, _H), Wxh, bh.reshape(1, _H),
                      W3, b3.reshape(1, _H), W4, b4.reshape(1, _D))
    return logits.reshape(_N)


# speedup vs baseline: 9.7405x; 1.0089x over previous
import functools

import jax
import jax.numpy as jnp
from jax import lax
from jax.experimental import pallas as pl
from jax.experimental.pallas import tpu as pltpu
from jax.experimental.pallas import tpu_sc as plsc

_N = 10000
_E = 320000
_D = 128
_H = 256
_CHUNK = 128
_NSUB = 16
_EP = 327680
_NCHUNKS = _EP // _CHUNK
_ACC_ROWS = 10240
_RPT = _ACC_ROWS // _NSUB
_G = 16


def _make_sc_agg(split_edges: bool, n_tables: int):
    chunks_per_core = _NCHUNKS // 2 if split_edges else _NCHUNKS
    cpt = chunks_per_core // _NSUB
    n_groups = cpt // _G
    mesh = plsc.VectorSubcoreMesh(core_axis_name="c", subcore_axis_name="s")

    @functools.partial(
        pl.kernel,
        mesh=mesh,
        out_type=jax.ShapeDtypeStruct((2, _N, _D), jnp.float32),
        scratch_types=[
            pltpu.VMEM((2, _G, _CHUNK), jnp.int32),
            pltpu.VMEM((2, _G, _CHUNK), jnp.int32),
            pltpu.VMEM((2, _CHUNK, _D), jnp.float32),
            pltpu.VMEM_SHARED((_ACC_ROWS, _D), jnp.float32),
            pltpu.SemaphoreType.DMA((2,)),
            pltpu.SemaphoreType.DMA,
        ],
    )
    def agg(table, src2d, dst2d, out, srcb, dstb, rows, acc, gsem, isem):
        c = lax.axis_index("c")
        s = lax.axis_index("s")
        tix = 0 if split_edges else c
        chunk0 = s * cpt + (c * chunks_per_core if split_edges else 0)

        def zero_body(i, carry):
            r = i // 8
            col = (i % 8) * 16
            rows[0, r, pl.ds(col, 16)] = jnp.zeros((16,), jnp.float32)
            return carry

        lax.fori_loop(0, _CHUNK * 8, zero_body, 0)
        for j in range(_RPT // _CHUNK):
            pltpu.sync_copy(rows.at[0],
                            acc.at[pl.ds(s * _RPT + j * _CHUNK, _CHUNK)])

        def start_idx(g):
            gb = lax.rem(g, 2)
            pltpu.async_copy(src2d.at[pl.ds(chunk0 + g * _G, _G)],
                             srcb.at[gb], isem)
            pltpu.async_copy(dst2d.at[pl.ds(chunk0 + g * _G, _G)],
                             dstb.at[gb], isem)

        def wait_idx(g):
            gb = lax.rem(g, 2)
            pltpu.make_async_copy(src2d.at[pl.ds(chunk0 + g * _G, _G)],
                                  srcb.at[gb], isem).wait()
            pltpu.make_async_copy(dst2d.at[pl.ds(chunk0 + g * _G, _G)],
                                  dstb.at[gb], isem).wait()

        def start_gather(i, b):
            g = i // _G
            pltpu.async_copy(
                table.at[tix].at[srcb.at[lax.rem(g, 2), i - g * _G]],
                rows.at[b], gsem.at[b])

        def wait_gather(i, b):
            g = i // _G
            pltpu.make_async_copy(
                table.at[tix].at[srcb.at[lax.rem(g, 2), i - g * _G]],
                rows.at[b], gsem.at[b]).wait()

        start_idx(0)
        wait_idx(0)
        start_gather(0, 0)

        plsc.subcore_barrier()

        def body(i, carry):
            g = i // _G
            j = i - g * _G
            b = lax.rem(i, 2)
            nxt = i + 1

            @pl.when(nxt < cpt)
            def _():
                @pl.when(j == _G - 1)
                def _():
                    wait_idx(g + 1)

                start_gather(nxt, 1 - b)

            @pl.when((j == 0) & (g + 1 < n_groups))
            def _():
                start_idx(g + 1)

            wait_gather(i, b)
            pltpu.sync_copy(rows.at[b],
                            acc.at[dstb.at[lax.rem(g, 2), i - g * _G]],
                            add=True)
            return carry

        lax.fori_loop(0, cpt, body, 0)

        plsc.subcore_barrier()

        @pl.when(s < _NSUB - 1)
        def _():
            r0 = s * _RPT
            pltpu.sync_copy(acc.at[pl.ds(r0, _RPT)],
                            out.at[c, pl.ds(r0, _RPT)])

        @pl.when(s == _NSUB - 1)
        def _():
            r0 = (_NSUB - 1) * _RPT
            nlast = _N - r0
            pltpu.sync_copy(acc.at[pl.ds(r0, nlast)],
                            out.at[c, pl.ds(r0, nlast)])

    return agg


_agg_edge_split = _make_sc_agg(split_edges=True, n_tables=1)
_agg_feat_split = _make_sc_agg(split_edges=False, n_tables=2)


_BLK = 1000


def _h1_body(x_ref, p_ref, w_ref, b_ref, o_ref):
    u = x_ref[...] + p_ref[0] + p_ref[1]
    h = jnp.dot(u, w_ref[...], preferred_element_type=jnp.float32) + b_ref[...]
    h = jnp.maximum(h, 0.0)
    o_ref[0] = h[:, :_D]
    o_ref[1] = h[:, _D:]


def _tc_h1(x, p, w1, b1):
    return pl.pallas_call(
        _h1_body,
        grid=(_N // _BLK,),
        in_specs=[
            pl.BlockSpec((_BLK, _D), lambda i: (i, 0)),
            pl.BlockSpec((2, _BLK, _D), lambda i: (0, i, 0)),
            pl.BlockSpec((_D, _H), lambda i: (0, 0)),
            pl.BlockSpec((1, _H), lambda i: (0, 0)),
        ],
        out_specs=pl.BlockSpec((2, _BLK, _D), lambda i: (0, i, 0)),
        out_shape=jax.ShapeDtypeStruct((2, _N, _D), jnp.float32),
    )(x, p, w1, b1)


_NB = _N // _BLK


def _tail_body(h_ref, a_ref, w2_ref, b2_ref, wxz_ref, bz_ref, wxh_ref, bh_ref,
               w3_ref, b3_ref, w4_ref, b4_ref, o_ref, z_ref, f_ref, zs_ref):
    i = pl.program_id(0)

    @pl.when(i < _NB)
    def _():
        ua = h_ref[0] + a_ref[0]
        ub = h_ref[1] + a_ref[1]
        z = (jnp.dot(ua, w2_ref[0], preferred_element_type=jnp.float32)
             + jnp.dot(ub, w2_ref[1], preferred_element_type=jnp.float32)
             + b2_ref[...])
        z_ref[pl.ds(i * _BLK, _BLK), :] = z

        @pl.when(i == 0)
        def _():
            f_ref[...] = jnp.zeros_like(f_ref)

        f_ref[...] += jnp.sum(z, axis=0, keepdims=True)

    @pl.when(i == _NB)
    def _():
        f = f_ref[...] * (1.0 / _N)
        zg = jax.nn.sigmoid(
            jnp.dot(f, wxz_ref[...], preferred_element_type=jnp.float32)
            + bz_ref[...])
        hh = jnp.tanh(
            jnp.dot(f, wxh_ref[...], preferred_element_type=jnp.float32)
            + bh_ref[...])
        ht = (1.0 - zg) * hh
        e = jnp.maximum(
            jnp.dot(ht, w3_ref[...], preferred_element_type=jnp.float32)
            + b3_ref[...], 0.0)
        zs_ref[...] = (jnp.dot(e, w4_ref[...], preferred_element_type=jnp.float32)
                       + b4_ref[...])

    @pl.when(i >= _NB)
    def _():
        k = i - _NB
        o_ref[...] = lax.dot_general(
            z_ref[pl.ds(k * _BLK, _BLK), :], zs_ref[...],
            dimension_numbers=(((1,), (1,)), ((), ())),
            preferred_element_type=jnp.float32)


def _tc_tail(h1pair, a2, w2, b2, wxz, bz, wxh, bh, w3, b3, w4, b4):
    def hspec():
        return pl.BlockSpec((2, _BLK, _D),
                            lambda i: (0, jnp.minimum(i, _NB - 1), 0))

    return pl.pallas_call(
        _tail_body,
        grid=(2 * _NB,),
        in_specs=[
            hspec(),
            hspec(),
            pl.BlockSpec((2, _D, _D), lambda i: (0, 0, 0)),
            pl.BlockSpec((1, _D), lambda i: (0, 0)),
            pl.BlockSpec((_D, _H), lambda i: (0, 0)),
            pl.BlockSpec((1, _H), lambda i: (0, 0)),
            pl.BlockSpec((_D, _H), lambda i: (0, 0)),
            pl.BlockSpec((1, _H), lambda i: (0, 0)),
            pl.BlockSpec((_H, _H), lambda i: (0, 0)),
            pl.BlockSpec((1, _H), lambda i: (0, 0)),
            pl.BlockSpec((_H, _D), lambda i: (0, 0)),
            pl.BlockSpec((1, _D), lambda i: (0, 0)),
        ],
        out_specs=pl.BlockSpec((_BLK, 1),
                               lambda i: (jnp.maximum(i - _NB, 0), 0)),
        out_shape=jax.ShapeDtypeStruct((_N, 1), jnp.float32),
        scratch_shapes=[
            pltpu.VMEM((_N, _D), jnp.float32),
            pltpu.VMEM((1, _D), jnp.float32),
            pltpu.VMEM((1, _D), jnp.float32),
        ],
    )(h1pair, a2, w2, b2, wxz, bz, wxh, bh, w3, b3, w4, b4)


def kernel(x, edge_index, W1, b1, W2, b2, Wxz, Whz, bz, Wxr, Whr, br,
           Wxh, Whh, bh, W3, b3, W4, b4):
    pad = _EP - _E
    ar = jnp.arange(pad, dtype=jnp.int32)
    src_pad = (ar * 37) % _N
    dst_pad = _N + (ar % (_ACC_ROWS - _N))
    src2d = jnp.concatenate([edge_index[0], src_pad]).reshape(_NCHUNKS, _CHUNK)
    dst2d = jnp.concatenate([edge_index[1], dst_pad]).reshape(_NCHUNKS, _CHUNK)

    p1 = _agg_edge_split(x.reshape(1, _N, _D), src2d, dst2d)
    h1pair = _tc_h1(x, p1, W1, b1.reshape(1, _H))
    a2 = _agg_feat_split(h1pair, src2d, dst2d)
    logits = _tc_tail(h1pair, a2, W2.reshape(2, _D, _D), b2.reshape(1, _D),
                      Wxz, bz.reshape(1, _H), Wxh, bh.reshape(1, _H),
                      W3, b3.reshape(1, _H), W4, b4.reshape(1, _D))
    return logits.reshape(_N)


# speedup vs baseline: 10.9006x; 1.1191x over previous
import functools

import jax
import jax.numpy as jnp
from jax import lax
from jax.experimental import pallas as pl
from jax.experimental.pallas import tpu as pltpu
from jax.experimental.pallas import tpu_sc as plsc

_N = 10000
_E = 320000
_D = 128
_H = 256
_CHUNK = 64
_NSUB = 16
_EP = 327680
_NCHUNKS = _EP // _CHUNK
_ACC_ROWS = 10240
_RPT = _ACC_ROWS // _NSUB
_G = 16
_K = 3
_NBUF = 4


def _make_sc_agg(split_edges: bool, n_tables: int):
    chunks_per_core = _NCHUNKS // 2 if split_edges else _NCHUNKS
    cpt = chunks_per_core // _NSUB
    n_groups = cpt // _G
    mesh = plsc.VectorSubcoreMesh(core_axis_name="c", subcore_axis_name="s")

    @functools.partial(
        pl.kernel,
        mesh=mesh,
        out_type=jax.ShapeDtypeStruct((2, _N, _D), jnp.float32),
        scratch_types=[
            pltpu.VMEM((2, _G, _CHUNK), jnp.int32),
            pltpu.VMEM((2, _G, _CHUNK), jnp.int32),
            pltpu.VMEM((_NBUF, _CHUNK, _D), jnp.float32),
            pltpu.VMEM_SHARED((_ACC_ROWS, _D), jnp.float32),
            pltpu.SemaphoreType.DMA((_NBUF,)),
            pltpu.SemaphoreType.DMA,
        ],
    )
    def agg(table, src2d, dst2d, out, srcb, dstb, rows, acc, gsem, isem):
        c = lax.axis_index("c")
        s = lax.axis_index("s")
        tix = 0 if split_edges else c
        chunk0 = s * cpt + (c * chunks_per_core if split_edges else 0)

        def zero_body(i, carry):
            r = i // 8
            col = (i % 8) * 16
            rows[0, r, pl.ds(col, 16)] = jnp.zeros((16,), jnp.float32)
            return carry

        lax.fori_loop(0, _CHUNK * 8, zero_body, 0)
        for j in range(_RPT // _CHUNK):
            pltpu.sync_copy(rows.at[0],
                            acc.at[pl.ds(s * _RPT + j * _CHUNK, _CHUNK)])

        def start_idx(g):
            gb = lax.rem(g, 2)
            pltpu.async_copy(src2d.at[pl.ds(chunk0 + g * _G, _G)],
                             srcb.at[gb], isem)
            pltpu.async_copy(dst2d.at[pl.ds(chunk0 + g * _G, _G)],
                             dstb.at[gb], isem)

        def wait_idx(g):
            gb = lax.rem(g, 2)
            pltpu.make_async_copy(src2d.at[pl.ds(chunk0 + g * _G, _G)],
                                  srcb.at[gb], isem).wait()
            pltpu.make_async_copy(dst2d.at[pl.ds(chunk0 + g * _G, _G)],
                                  dstb.at[gb], isem).wait()

        def start_gather(i, b):
            g = i // _G
            pltpu.async_copy(
                table.at[tix].at[srcb.at[lax.rem(g, 2), i - g * _G]],
                rows.at[b], gsem.at[b])

        def wait_gather(i, b):
            g = i // _G
            pltpu.make_async_copy(
                table.at[tix].at[srcb.at[lax.rem(g, 2), i - g * _G]],
                rows.at[b], gsem.at[b]).wait()

        start_idx(0)
        wait_idx(0)
        for k in range(_K):
            start_gather(k, k)

        plsc.subcore_barrier()

        def body(i, carry):
            g = i // _G
            j = i - g * _G
            b = lax.rem(i, _NBUF)
            nxt = i + _K

            @pl.when(nxt < cpt)
            def _():
                ng = nxt // _G

                @pl.when(nxt == ng * _G)
                def _():
                    wait_idx(ng)

                start_gather(nxt, lax.rem(nxt, _NBUF))

            @pl.when((j == 0) & (g + 1 < n_groups))
            def _():
                start_idx(g + 1)

            wait_gather(i, b)
            pltpu.sync_copy(rows.at[b],
                            acc.at[dstb.at[lax.rem(g, 2), i - g * _G]],
                            add=True)
            return carry

        lax.fori_loop(0, cpt, body, 0)

        plsc.subcore_barrier()

        @pl.when(s < _NSUB - 1)
        def _():
            r0 = s * _RPT
            pltpu.sync_copy(acc.at[pl.ds(r0, _RPT)],
                            out.at[c, pl.ds(r0, _RPT)])

        @pl.when(s == _NSUB - 1)
        def _():
            r0 = (_NSUB - 1) * _RPT
            nlast = _N - r0
            pltpu.sync_copy(acc.at[pl.ds(r0, nlast)],
                            out.at[c, pl.ds(r0, nlast)])

    return agg


_agg_edge_split = _make_sc_agg(split_edges=True, n_tables=1)
_agg_feat_split = _make_sc_agg(split_edges=False, n_tables=2)


_BLK = 1000


def _h1_body(x_ref, p_ref, w_ref, b_ref, o_ref):
    u = x_ref[...] + p_ref[0] + p_ref[1]
    h = jnp.dot(u, w_ref[...], preferred_element_type=jnp.float32) + b_ref[...]
    h = jnp.maximum(h, 0.0)
    o_ref[0] = h[:, :_D]
    o_ref[1] = h[:, _D:]


def _tc_h1(x, p, w1, b1):
    return pl.pallas_call(
        _h1_body,
        grid=(_N // _BLK,),
        in_specs=[
            pl.BlockSpec((_BLK, _D), lambda i: (i, 0)),
            pl.BlockSpec((2, _BLK, _D), lambda i: (0, i, 0)),
            pl.BlockSpec((_D, _H), lambda i: (0, 0)),
            pl.BlockSpec((1, _H), lambda i: (0, 0)),
        ],
        out_specs=pl.BlockSpec((2, _BLK, _D), lambda i: (0, i, 0)),
        out_shape=jax.ShapeDtypeStruct((2, _N, _D), jnp.float32),
    )(x, p, w1, b1)


_NB = _N // _BLK


def _tail_body(h_ref, a_ref, w2_ref, b2_ref, wxz_ref, bz_ref, wxh_ref, bh_ref,
               w3_ref, b3_ref, w4_ref, b4_ref, o_ref, z_ref, f_ref, zs_ref):
    i = pl.program_id(0)

    @pl.when(i < _NB)
    def _():
        ua = h_ref[0] + a_ref[0]
        ub = h_ref[1] + a_ref[1]
        z = (jnp.dot(ua, w2_ref[0], preferred_element_type=jnp.float32)
             + jnp.dot(ub, w2_ref[1], preferred_element_type=jnp.float32)
             + b2_ref[...])
        z_ref[pl.ds(i * _BLK, _BLK), :] = z

        @pl.when(i == 0)
        def _():
            f_ref[...] = jnp.zeros_like(f_ref)

        f_ref[...] += jnp.sum(z, axis=0, keepdims=True)

    @pl.when(i == _NB)
    def _():
        f = f_ref[...] * (1.0 / _N)
        zg = jax.nn.sigmoid(
            jnp.dot(f, wxz_ref[...], preferred_element_type=jnp.float32)
            + bz_ref[...])
        hh = jnp.tanh(
            jnp.dot(f, wxh_ref[...], preferred_element_type=jnp.float32)
            + bh_ref[...])
        ht = (1.0 - zg) * hh
        e = jnp.maximum(
            jnp.dot(ht, w3_ref[...], preferred_element_type=jnp.float32)
            + b3_ref[...], 0.0)
        zs_ref[...] = (jnp.dot(e, w4_ref[...], preferred_element_type=jnp.float32)
                       + b4_ref[...])

    @pl.when(i >= _NB)
    def _():
        k = i - _NB
        o_ref[...] = lax.dot_general(
            z_ref[pl.ds(k * _BLK, _BLK), :], zs_ref[...],
            dimension_numbers=(((1,), (1,)), ((), ())),
            preferred_element_type=jnp.float32)


def _tc_tail(h1pair, a2, w2, b2, wxz, bz, wxh, bh, w3, b3, w4, b4):
    def hspec():
        return pl.BlockSpec((2, _BLK, _D),
                            lambda i: (0, jnp.minimum(i, _NB - 1), 0))

    return pl.pallas_call(
        _tail_body,
        grid=(2 * _NB,),
        in_specs=[
            hspec(),
            hspec(),
            pl.BlockSpec((2, _D, _D), lambda i: (0, 0, 0)),
            pl.BlockSpec((1, _D), lambda i: (0, 0)),
            pl.BlockSpec((_D, _H), lambda i: (0, 0)),
            pl.BlockSpec((1, _H), lambda i: (0, 0)),
            pl.BlockSpec((_D, _H), lambda i: (0, 0)),
            pl.BlockSpec((1, _H), lambda i: (0, 0)),
            pl.BlockSpec((_H, _H), lambda i: (0, 0)),
            pl.BlockSpec((1, _H), lambda i: (0, 0)),
            pl.BlockSpec((_H, _D), lambda i: (0, 0)),
            pl.BlockSpec((1, _D), lambda i: (0, 0)),
        ],
        out_specs=pl.BlockSpec((_BLK, 1),
                               lambda i: (jnp.maximum(i - _NB, 0), 0)),
        out_shape=jax.ShapeDtypeStruct((_N, 1), jnp.float32),
        scratch_shapes=[
            pltpu.VMEM((_N, _D), jnp.float32),
            pltpu.VMEM((1, _D), jnp.float32),
            pltpu.VMEM((1, _D), jnp.float32),
        ],
    )(h1pair, a2, w2, b2, wxz, bz, wxh, bh, w3, b3, w4, b4)


def kernel(x, edge_index, W1, b1, W2, b2, Wxz, Whz, bz, Wxr, Whr, br,
           Wxh, Whh, bh, W3, b3, W4, b4):
    pad = _EP - _E
    ar = jnp.arange(pad, dtype=jnp.int32)
    src_pad = (ar * 37) % _N
    dst_pad = _N + (ar % (_ACC_ROWS - _N))
    src2d = jnp.concatenate([edge_index[0], src_pad]).reshape(_NCHUNKS, _CHUNK)
    dst2d = jnp.concatenate([edge_index[1], dst_pad]).reshape(_NCHUNKS, _CHUNK)

    p1 = _agg_edge_split(x.reshape(1, _N, _D), src2d, dst2d)
    h1pair = _tc_h1(x, p1, W1, b1.reshape(1, _H))
    a2 = _agg_feat_split(h1pair, src2d, dst2d)
    logits = _tc_tail(h1pair, a2, W2.reshape(2, _D, _D), b2.reshape(1, _D),
                      Wxz, bz.reshape(1, _H), Wxh, bh.reshape(1, _H),
                      W3, b3.reshape(1, _H), W4, b4.reshape(1, _D))
    return logits.reshape(_N)


# speedup vs baseline: 11.0398x; 1.0128x over previous
import functools

import jax
import jax.numpy as jnp
from jax import lax
from jax.experimental import pallas as pl
from jax.experimental.pallas import tpu as pltpu
from jax.experimental.pallas import tpu_sc as plsc

_N = 10000
_E = 320000
_D = 128
_H = 256
_CHUNK = 64
_NSUB = 16
_EP = 327680
_NCHUNKS = _EP // _CHUNK
_ACC_ROWS = 10240
_RPT = _ACC_ROWS // _NSUB
_G = 16
_K = 4
_NBUF = 5


def _make_sc_agg(split_edges: bool, n_tables: int):
    chunks_per_core = _NCHUNKS // 2 if split_edges else _NCHUNKS
    cpt = chunks_per_core // _NSUB
    n_groups = cpt // _G
    mesh = plsc.VectorSubcoreMesh(core_axis_name="c", subcore_axis_name="s")

    @functools.partial(
        pl.kernel,
        mesh=mesh,
        out_type=jax.ShapeDtypeStruct((2, _N, _D), jnp.float32),
        scratch_types=[
            pltpu.VMEM((2, _G, _CHUNK), jnp.int32),
            pltpu.VMEM((2, _G, _CHUNK), jnp.int32),
            pltpu.VMEM((_NBUF, _CHUNK, _D), jnp.float32),
            pltpu.VMEM_SHARED((_ACC_ROWS, _D), jnp.float32),
            pltpu.SemaphoreType.DMA((_NBUF,)),
            pltpu.SemaphoreType.DMA,
        ],
    )
    def agg(table, src2d, dst2d, out, srcb, dstb, rows, acc, gsem, isem):
        c = lax.axis_index("c")
        s = lax.axis_index("s")
        tix = 0 if split_edges else c
        chunk0 = s * cpt + (c * chunks_per_core if split_edges else 0)

        def zero_body(i, carry):
            r = i // 8
            col = (i % 8) * 16
            rows[0, r, pl.ds(col, 16)] = jnp.zeros((16,), jnp.float32)
            return carry

        lax.fori_loop(0, _CHUNK * 8, zero_body, 0)
        for j in range(_RPT // _CHUNK):
            pltpu.sync_copy(rows.at[0],
                            acc.at[pl.ds(s * _RPT + j * _CHUNK, _CHUNK)])

        def start_idx(g):
            gb = lax.rem(g, 2)
            pltpu.async_copy(src2d.at[pl.ds(chunk0 + g * _G, _G)],
                             srcb.at[gb], isem)
            pltpu.async_copy(dst2d.at[pl.ds(chunk0 + g * _G, _G)],
                             dstb.at[gb], isem)

        def wait_idx(g):
            gb = lax.rem(g, 2)
            pltpu.make_async_copy(src2d.at[pl.ds(chunk0 + g * _G, _G)],
                                  srcb.at[gb], isem).wait()
            pltpu.make_async_copy(dst2d.at[pl.ds(chunk0 + g * _G, _G)],
                                  dstb.at[gb], isem).wait()

        def start_gather(i, b):
            g = i // _G
            pltpu.async_copy(
                table.at[tix].at[srcb.at[lax.rem(g, 2), i - g * _G]],
                rows.at[b], gsem.at[b])

        def wait_gather(i, b):
            g = i // _G
            pltpu.make_async_copy(
                table.at[tix].at[srcb.at[lax.rem(g, 2), i - g * _G]],
                rows.at[b], gsem.at[b]).wait()

        start_idx(0)
        wait_idx(0)
        for k in range(_K):
            start_gather(k, k)

        plsc.subcore_barrier()

        def body(i, carry):
            g = i // _G
            j = i - g * _G
            b = lax.rem(i, _NBUF)
            nxt = i + _K

            @pl.when(nxt < cpt)
            def _():
                ng = nxt // _G

                @pl.when(nxt == ng * _G)
                def _():
                    wait_idx(ng)

                start_gather(nxt, lax.rem(nxt, _NBUF))

            @pl.when((j == 0) & (g + 1 < n_groups))
            def _():
                start_idx(g + 1)

            wait_gather(i, b)
            pltpu.sync_copy(rows.at[b],
                            acc.at[dstb.at[lax.rem(g, 2), i - g * _G]],
                            add=True)
            return carry

        lax.fori_loop(0, cpt, body, 0)

        plsc.subcore_barrier()

        @pl.when(s < _NSUB - 1)
        def _():
            r0 = s * _RPT
            pltpu.sync_copy(acc.at[pl.ds(r0, _RPT)],
                            out.at[c, pl.ds(r0, _RPT)])

        @pl.when(s == _NSUB - 1)
        def _():
            r0 = (_NSUB - 1) * _RPT
            nlast = _N - r0
            pltpu.sync_copy(acc.at[pl.ds(r0, nlast)],
                            out.at[c, pl.ds(r0, nlast)])

    return agg


_agg_edge_split = _make_sc_agg(split_edges=True, n_tables=1)
_agg_feat_split = _make_sc_agg(split_edges=False, n_tables=2)


_BLK = 1000


def _h1_body(x_ref, p_ref, w_ref, b_ref, o_ref):
    u = x_ref[...] + p_ref[0] + p_ref[1]
    h = jnp.dot(u, w_ref[...], preferred_element_type=jnp.float32) + b_ref[...]
    h = jnp.maximum(h, 0.0)
    o_ref[0] = h[:, :_D]
    o_ref[1] = h[:, _D:]


def _tc_h1(x, p, w1, b1):
    return pl.pallas_call(
        _h1_body,
        grid=(_N // _BLK,),
        in_specs=[
            pl.BlockSpec((_BLK, _D), lambda i: (i, 0)),
            pl.BlockSpec((2, _BLK, _D), lambda i: (0, i, 0)),
            pl.BlockSpec((_D, _H), lambda i: (0, 0)),
            pl.BlockSpec((1, _H), lambda i: (0, 0)),
        ],
        out_specs=pl.BlockSpec((2, _BLK, _D), lambda i: (0, i, 0)),
        out_shape=jax.ShapeDtypeStruct((2, _N, _D), jnp.float32),
    )(x, p, w1, b1)


_NB = _N // _BLK


def _tail_body(h_ref, a_ref, w2_ref, b2_ref, wxz_ref, bz_ref, wxh_ref, bh_ref,
               w3_ref, b3_ref, w4_ref, b4_ref, o_ref, z_ref, f_ref, zs_ref):
    i = pl.program_id(0)

    @pl.when(i < _NB)
    def _():
        ua = h_ref[0] + a_ref[0]
        ub = h_ref[1] + a_ref[1]
        z = (jnp.dot(ua, w2_ref[0], preferred_element_type=jnp.float32)
             + jnp.dot(ub, w2_ref[1], preferred_element_type=jnp.float32)
             + b2_ref[...])
        z_ref[pl.ds(i * _BLK, _BLK), :] = z

        @pl.when(i == 0)
        def _():
            f_ref[...] = jnp.zeros_like(f_ref)

        f_ref[...] += jnp.sum(z, axis=0, keepdims=True)

    @pl.when(i == _NB)
    def _():
        f = f_ref[...] * (1.0 / _N)
        zg = jax.nn.sigmoid(
            jnp.dot(f, wxz_ref[...], preferred_element_type=jnp.float32)
            + bz_ref[...])
        hh = jnp.tanh(
            jnp.dot(f, wxh_ref[...], preferred_element_type=jnp.float32)
            + bh_ref[...])
        ht = (1.0 - zg) * hh
        e = jnp.maximum(
            jnp.dot(ht, w3_ref[...], preferred_element_type=jnp.float32)
            + b3_ref[...], 0.0)
        zs_ref[...] = (jnp.dot(e, w4_ref[...], preferred_element_type=jnp.float32)
                       + b4_ref[...])

    @pl.when(i >= _NB)
    def _():
        k = i - _NB
        o_ref[...] = lax.dot_general(
            z_ref[pl.ds(k * _BLK, _BLK), :], zs_ref[...],
            dimension_numbers=(((1,), (1,)), ((), ())),
            preferred_element_type=jnp.float32)


def _tc_tail(h1pair, a2, w2, b2, wxz, bz, wxh, bh, w3, b3, w4, b4):
    def hspec():
        return pl.BlockSpec((2, _BLK, _D),
                            lambda i: (0, jnp.minimum(i, _NB - 1), 0))

    return pl.pallas_call(
        _tail_body,
        grid=(2 * _NB,),
        in_specs=[
            hspec(),
            hspec(),
            pl.BlockSpec((2, _D, _D), lambda i: (0, 0, 0)),
            pl.BlockSpec((1, _D), lambda i: (0, 0)),
            pl.BlockSpec((_D, _H), lambda i: (0, 0)),
            pl.BlockSpec((1, _H), lambda i: (0, 0)),
            pl.BlockSpec((_D, _H), lambda i: (0, 0)),
            pl.BlockSpec((1, _H), lambda i: (0, 0)),
            pl.BlockSpec((_H, _H), lambda i: (0, 0)),
            pl.BlockSpec((1, _H), lambda i: (0, 0)),
            pl.BlockSpec((_H, _D), lambda i: (0, 0)),
            pl.BlockSpec((1, _D), lambda i: (0, 0)),
        ],
        out_specs=pl.BlockSpec((_BLK, 1),
                               lambda i: (jnp.maximum(i - _NB, 0), 0)),
        out_shape=jax.ShapeDtypeStruct((_N, 1), jnp.float32),
        scratch_shapes=[
            pltpu.VMEM((_N, _D), jnp.float32),
            pltpu.VMEM((1, _D), jnp.float32),
            pltpu.VMEM((1, _D), jnp.float32),
        ],
    )(h1pair, a2, w2, b2, wxz, bz, wxh, bh, w3, b3, w4, b4)


def kernel(x, edge_index, W1, b1, W2, b2, Wxz, Whz, bz, Wxr, Whr, br,
           Wxh, Whh, bh, W3, b3, W4, b4):
    pad = _EP - _E
    ar = jnp.arange(pad, dtype=jnp.int32)
    src_pad = (ar * 37) % _N
    dst_pad = _N + (ar % (_ACC_ROWS - _N))
    src2d = jnp.concatenate([edge_index[0], src_pad]).reshape(_NCHUNKS, _CHUNK)
    dst2d = jnp.concatenate([edge_index[1], dst_pad]).reshape(_NCHUNKS, _CHUNK)

    p1 = _agg_edge_split(x.reshape(1, _N, _D), src2d, dst2d)
    h1pair = _tc_h1(x, p1, W1, b1.reshape(1, _H))
    a2 = _agg_feat_split(h1pair, src2d, dst2d)
    logits = _tc_tail(h1pair, a2, W2.reshape(2, _D, _D), b2.reshape(1, _D),
                      Wxz, bz.reshape(1, _H), Wxh, bh.reshape(1, _H),
                      W3, b3.reshape(1, _H), W4, b4.reshape(1, _D))
    return logits.reshape(_N)
